# baseline (device time: 139527 ns/iter reference)
import jax
import jax.numpy as jnp
from jax import lax
from jax.experimental import pallas as pl
from jax.experimental.pallas import tpu as pltpu

T_CORR = 64


def kernel(x, A, B, C):
    Bsz, S, D = x.shape
    N = A.shape[1]

    def body(x_ref, A_ref, B_ref, C_ref, out_ref, h_ref, hin_ref,
             send_sem, recv_sem):
        my_x = lax.axis_index("x")
        my_y = lax.axis_index("y")
        nbr = (my_x, 1 - my_y)

        barrier = pltpu.get_barrier_semaphore()
        pl.semaphore_signal(barrier, inc=1, device_id=nbr,
                            device_id_type=pl.DeviceIdType.MESH)
        pl.semaphore_wait(barrier, 1)

        dAT = jnp.exp(A_ref[...]).T

        def step(t, h):
            xt = x_ref[:, pl.ds(t, 1), :]
            Bt = jnp.swapaxes(B_ref[:, pl.ds(t, 1), :], 1, 2)
            Ct = jnp.swapaxes(C_ref[:, pl.ds(t, 1), :], 1, 2)
            h = h * dAT[None] + xt * Bt
            out_ref[:, pl.ds(t, 1), :] = jnp.sum(h * Ct, axis=1,
                                                 keepdims=True)
            return h
        h = lax.fori_loop(0, S, step,
                          jnp.zeros((Bsz, N, D), jnp.float32))
        h_ref[...] = h

        copy = pltpu.make_async_remote_copy(
            src_ref=h_ref, dst_ref=hin_ref,
            send_sem=send_sem, recv_sem=recv_sem,
            device_id=nbr, device_id_type=pl.DeviceIdType.MESH,
        )

        @pl.when(my_y == 0)
        def _():
            copy.start()
            copy.wait_send()

        @pl.when(my_y == 1)
        def _():
            copy.wait_recv()

            def corr_step(t, g):
                g = g * dAT[None]
                Ct = jnp.swapaxes(C_ref[:, pl.ds(t, 1), :], 1, 2)
                yc = jnp.sum(g * Ct, axis=1, keepdims=True)
                out_ref[:, pl.ds(t, 1), :] = out_ref[:, pl.ds(t, 1), :] + yc
                return g
            lax.fori_loop(0, T_CORR, corr_step, hin_ref[...])

    return pl.pallas_call(
        body,
        out_shape=jax.ShapeDtypeStruct((Bsz, S, D), jnp.float32),
        in_specs=[pl.BlockSpec(memory_space=pltpu.VMEM)] * 4,
        out_specs=pl.BlockSpec(memory_space=pltpu.VMEM),
        scratch_shapes=[
            pltpu.VMEM((Bsz, N, D), jnp.float32),
            pltpu.VMEM((Bsz, N, D), jnp.float32),
            pltpu.SemaphoreType.DMA,
            pltpu.SemaphoreType.DMA,
        ],
        compiler_params=pltpu.CompilerParams(collective_id=0),
    )(x, A, B, C)


# device time: 71064 ns/iter; 1.9634x vs baseline; 1.9634x over previous
import jax
import jax.numpy as jnp
from jax import lax
from jax.experimental import pallas as pl
from jax.experimental.pallas import tpu as pltpu

T_CORR = 64
L = 64


def kernel(x, A, B, C):
    Bsz, S, D = x.shape
    N = A.shape[1]

    def body(x_ref, A_ref, B_ref, C_ref, out_ref, h_ref, hin_ref,
             u_ref, send_sem, recv_sem):
        my_x = lax.axis_index("x")
        my_y = lax.axis_index("y")
        nbr = (my_x, 1 - my_y)

        barrier = pltpu.get_barrier_semaphore()
        pl.semaphore_signal(barrier, inc=1, device_id=nbr,
                            device_id_type=pl.DeviceIdType.MESH)
        pl.semaphore_wait(barrier, 1)

        dAT = jnp.exp(A_ref[...]).T

        h = jnp.zeros((Bsz, N, D), jnp.float32)
        for c in range(S // L):
            lo = c * L
            xc = x_ref[:, lo:lo + L, :]
            bc = B_ref[:, lo:lo + L, :]
            u_ref[...] = xc[:, :, None, :] * bc[..., None]

            def step(j, h):
                h = h * dAT[None] + u_ref[:, j]
                u_ref[:, j] = h
                return h
            h = lax.fori_loop(0, L, step, h, unroll=4)

            cc = C_ref[:, lo:lo + L, :]
            out_ref[:, lo:lo + L, :] = jnp.sum(
                u_ref[...] * cc[..., None], axis=2)
        h_ref[...] = h

        copy = pltpu.make_async_remote_copy(
            src_ref=h_ref, dst_ref=hin_ref,
            send_sem=send_sem, recv_sem=recv_sem,
            device_id=nbr, device_id_type=pl.DeviceIdType.MESH,
        )

        @pl.when(my_y == 0)
        def _():
            copy.start()
            copy.wait_send()

        @pl.when(my_y == 1)
        def _():
            copy.wait_recv()
            AT = A_ref[...].T
            jp1 = (lax.broadcasted_iota(
                jnp.int32, (T_CORR, N, D), 0) + 1).astype(jnp.float32)
            E = jnp.exp(AT[None] * jp1)
            for b in range(Bsz):
                cb = C_ref[b, :T_CORR, :]
                corr = jnp.sum(
                    E * hin_ref[b][None] * cb[..., None], axis=1)
                out_ref[b, :T_CORR, :] = out_ref[b, :T_CORR, :] + corr

    return pl.pallas_call(
        body,
        out_shape=jax.ShapeDtypeStruct((Bsz, S, D), jnp.float32),
        in_specs=[pl.BlockSpec(memory_space=pltpu.VMEM)] * 4,
        out_specs=pl.BlockSpec(memory_space=pltpu.VMEM),
        scratch_shapes=[
            pltpu.VMEM((Bsz, N, D), jnp.float32),
            pltpu.VMEM((Bsz, N, D), jnp.float32),
            pltpu.VMEM((Bsz, L, N, D), jnp.float32),
            pltpu.SemaphoreType.DMA,
            pltpu.SemaphoreType.DMA,
        ],
        compiler_params=pltpu.CompilerParams(collective_id=0),
    )(x, A, B, C)
